# baseline (speedup 1.0000x reference)
import jax
import jax.numpy as jnp
from jax import lax
from jax.experimental import pallas as pl
from jax.experimental.pallas import tpu as pltpu
from jax.experimental.pallas import tpu_sc as plsc

N = 10000
D = 128
E = 320000

NC = 2
NS = 16
CHUNK = 128
NPAD = 10240
EPAD = 327680
KB = 10
IT0 = 30
IT1 = 130
ROWS_PER_TILE = NPAD // NS

_HIGH = lax.Precision.HIGHEST


def _sc_agg_body(h_hbm, row_hbm, col_hbm, out_hbm,
                 cv0, cv1, rv0, rv1, gb0, gb1, acc,
                 sic0, sic1, sir0, sir1, sg0, sg1):
    c = lax.axis_index("core")
    s = lax.axis_index("subcore")

    colv = (cv0, cv1)
    rowv = (rv0, rv1)
    gbufs = (gb0, gb1)
    sic = (sic0, sic1)
    sir = (sir0, sir1)
    sgs = (sg0, sg1)

    base0 = s * (IT0 * CHUNK)
    base1 = NS * IT0 * CHUNK + s * (IT1 * CHUNK)

    def ic_start(off, j, b):
        pltpu.async_copy(col_hbm.at[pl.ds(off + j * CHUNK, CHUNK)], colv[b], sic[b])

    def ic_wait(off, j, b):
        pltpu.make_async_copy(col_hbm.at[pl.ds(off + j * CHUNK, CHUNK)], colv[b], sic[b]).wait()

    def ir_start(off, j, b):
        pltpu.async_copy(row_hbm.at[pl.ds(off + j * CHUNK, CHUNK)], rowv[b], sir[b])

    def ir_wait(off, j, b):
        pltpu.make_async_copy(row_hbm.at[pl.ds(off + j * CHUNK, CHUNK)], rowv[b], sir[b]).wait()

    def g_start(b):
        pltpu.async_copy(h_hbm.at[colv[b]], gbufs[b], sgs[b])

    def g_wait(b):
        pltpu.make_async_copy(h_hbm.at[colv[b]], gbufs[b], sgs[b]).wait()

    @pl.loop(0, CHUNK)
    def _(r):
        @pl.loop(0, D, step=16)
        def _(j):
            gb0[r, pl.ds(j, 16)] = jnp.zeros((16,), jnp.float32)

    @pl.loop(0, ROWS_PER_TILE, step=CHUNK)
    def _(k):
        pltpu.sync_copy(gb0, acc.at[pl.ds(s * ROWS_PER_TILE + k, CHUNK)])

    plsc.subcore_barrier()

    def agg_loop(base, iters):
        @pl.loop(0, iters, step=KB)
        def _(g):
            off = base + g * CHUNK
            for b in range(2):
                ic_start(off, b, b)
                ir_start(off, b, b)
            for b in range(2):
                ic_wait(off, b, b)
                g_start(b)
            for j in range(KB):
                b = j % 2
                g_wait(b)
                if j + 2 < KB:
                    ic_start(off, j + 2, b)
                ir_wait(off, j, b)
                pltpu.sync_copy(gbufs[b], acc.at[rowv[b]], add=True)
                if j + 2 < KB:
                    ir_start(off, j + 2, b)
                    ic_wait(off, j + 2, b)
                    g_start(b)

    @pl.when(c == 0)
    def _():
        agg_loop(base0, IT0)

    @pl.when(c == 1)
    def _():
        agg_loop(base1, IT1)

    plsc.subcore_barrier()

    pltpu.sync_copy(acc.at[pl.ds(s * ROWS_PER_TILE, ROWS_PER_TILE)],
                    out_hbm.at[c, pl.ds(s * ROWS_PER_TILE, ROWS_PER_TILE)])


def _sc_aggregate(h, rowp, colp):
    mesh = plsc.VectorSubcoreMesh(core_axis_name="core", subcore_axis_name="subcore")
    k = pl.kernel(
        _sc_agg_body,
        out_type=jax.ShapeDtypeStruct((NC, NPAD, D), jnp.float32),
        mesh=mesh,
        scratch_types=(
            [pltpu.VMEM((CHUNK,), jnp.int32)] * 4
            + [pltpu.VMEM((CHUNK, D), jnp.float32)] * 2
            + [pltpu.VMEM_SHARED((NPAD, D), jnp.float32)]
            + [pltpu.SemaphoreType.DMA] * 6
        ),
    )
    return k(h, rowp, colp)


BLK = 1000
GRID = N // BLK


def _stage_a_body(x_ref, wt_ref, bt_ref, w1_ref, xt_ref, h1_ref):
    xt = jnp.dot(x_ref[...], wt_ref[...], precision=_HIGH,
                 preferred_element_type=jnp.float32) + bt_ref[...]
    xt_ref[...] = xt
    h1_ref[...] = jnp.dot(xt, w1_ref[...], precision=_HIGH,
                          preferred_element_type=jnp.float32)


def _stage_mid_body(xp_ref, p_ref, b_ref, w_ref, xn_ref, hn_ref):
    agg = p_ref[0] + p_ref[1] + b_ref[...]
    xn = xp_ref[...] + jnp.maximum(agg, 0.0)
    xn_ref[...] = xn
    hn_ref[...] = jnp.dot(xn, w_ref[...], precision=_HIGH,
                          preferred_element_type=jnp.float32)


def _stage_out_body(xp_ref, p_ref, b_ref, o_ref):
    agg = p_ref[0] + p_ref[1] + b_ref[...]
    o_ref[...] = xp_ref[...] + jnp.maximum(agg, 0.0)


_row_spec = pl.BlockSpec((BLK, D), lambda i: (i, 0))
_mat_spec = pl.BlockSpec((D, D), lambda i: (0, 0))
_vec_spec = pl.BlockSpec((1, D), lambda i: (0, 0))
_par_spec = pl.BlockSpec((NC, BLK, D), lambda i: (0, i, 0))
_rowD = jax.ShapeDtypeStruct((N, D), jnp.float32)


def _stage_a(x, wt, bt, w1):
    return pl.pallas_call(
        _stage_a_body,
        grid=(GRID,),
        in_specs=[_row_spec, _mat_spec, _vec_spec, _mat_spec],
        out_specs=[_row_spec, _row_spec],
        out_shape=[_rowD, _rowD],
    )(x, wt, bt, w1)


def _stage_mid(xp, p, b, w):
    return pl.pallas_call(
        _stage_mid_body,
        grid=(GRID,),
        in_specs=[_row_spec, _par_spec, _vec_spec, _mat_spec],
        out_specs=[_row_spec, _row_spec],
        out_shape=[_rowD, _rowD],
    )(xp, p, b, w)


def _stage_out(xp, p, b):
    return pl.pallas_call(
        _stage_out_body,
        grid=(GRID,),
        in_specs=[_row_spec, _par_spec, _vec_spec],
        out_specs=_row_spec,
        out_shape=_rowD,
    )(xp, p, b)


@jax.jit
def kernel(x, edge_index, Wt, bt, W1, b1, W2, b2):
    row = edge_index[0]
    col = edge_index[1]
    npad = EPAD - E
    rowp = jnp.concatenate([row, jnp.full((npad,), N, jnp.int32)])
    colp = jnp.concatenate([col, jnp.zeros((npad,), jnp.int32)])
    bt2 = bt.reshape(1, D)
    b12 = b1.reshape(1, D)
    b22 = b2.reshape(1, D)

    xt, h1 = _stage_a(x, Wt, bt2, W1)
    p1 = _sc_aggregate(h1, rowp, colp)
    x1, h2 = _stage_mid(xt, p1, b12, W2)
    p2 = _sc_aggregate(h2, rowp, colp)
    out = _stage_out(x1, p2, b22)
    return (out, jnp.float32(0.0))


# speedup vs baseline: 1.0697x; 1.0697x over previous
import jax
import jax.numpy as jnp
from jax import lax
from jax.experimental import pallas as pl
from jax.experimental.pallas import tpu as pltpu
from jax.experimental.pallas import tpu_sc as plsc

N = 10000
D = 128
E = 320000

NC = 2
NS = 16
CHUNK = 128
NPAD = 10240
EPAD = 327680
ITERS = EPAD // (NC * NS * CHUNK)
KB = 16
NB = ITERS // KB
ROWS_PER_TILE = NPAD // NS
E_PER_SC = EPAD // NC
E_PER_TILE = EPAD // (NC * NS)

_HIGH = lax.Precision.HIGHEST


def _sc_agg_body(h_hbm, row_hbm, col_hbm, out_hbm,
                 cv0, cv1, rv0, rv1, gb0, gb1, acc,
                 sic0, sic1, sir0, sir1, sg0, sg1):
    c = lax.axis_index("core")
    s = lax.axis_index("subcore")

    colv = (cv0, cv1)
    rowv = (rv0, rv1)
    gbufs = (gb0, gb1)
    sic = (sic0, sic1)
    sir = (sir0, sir1)
    sgs = (sg0, sg1)

    base = c * E_PER_SC + s * E_PER_TILE

    def ic_start(off, j, b):
        pltpu.async_copy(col_hbm.at[pl.ds(off + j * CHUNK, CHUNK)], colv[b], sic[b])

    def ic_wait(off, j, b):
        pltpu.make_async_copy(col_hbm.at[pl.ds(off + j * CHUNK, CHUNK)], colv[b], sic[b]).wait()

    def ir_start(off, j, b):
        pltpu.async_copy(row_hbm.at[pl.ds(off + j * CHUNK, CHUNK)], rowv[b], sir[b])

    def ir_wait(off, j, b):
        pltpu.make_async_copy(row_hbm.at[pl.ds(off + j * CHUNK, CHUNK)], rowv[b], sir[b]).wait()

    def g_start(b):
        pltpu.async_copy(h_hbm.at[colv[b]], gbufs[b], sgs[b])

    def g_wait(b):
        pltpu.make_async_copy(h_hbm.at[colv[b]], gbufs[b], sgs[b]).wait()

    @pl.loop(0, CHUNK)
    def _(r):
        @pl.loop(0, D, step=16)
        def _(j):
            gb0[r, pl.ds(j, 16)] = jnp.zeros((16,), jnp.float32)

    @pl.loop(0, ROWS_PER_TILE, step=CHUNK)
    def _(k):
        pltpu.sync_copy(gb0, acc.at[pl.ds(s * ROWS_PER_TILE + k, CHUNK)])

    plsc.subcore_barrier()

    @pl.loop(0, ITERS, step=KB)
    def _(g):
        off = base + g * CHUNK
        for b in range(2):
            ic_start(off, b, b)
            ir_start(off, b, b)
        for b in range(2):
            ic_wait(off, b, b)
            g_start(b)
        for j in range(KB):
            b = j % 2
            g_wait(b)
            if j + 2 < KB:
                ic_start(off, j + 2, b)
            ir_wait(off, j, b)
            pltpu.sync_copy(gbufs[b], acc.at[rowv[b]], add=True)
            if j + 2 < KB:
                ir_start(off, j + 2, b)
                ic_wait(off, j + 2, b)
                g_start(b)

    plsc.subcore_barrier()

    pltpu.sync_copy(acc.at[pl.ds(s * ROWS_PER_TILE, ROWS_PER_TILE)],
                    out_hbm.at[c, pl.ds(s * ROWS_PER_TILE, ROWS_PER_TILE)])


def _sc_aggregate(h, rowp, colp):
    mesh = plsc.VectorSubcoreMesh(core_axis_name="core", subcore_axis_name="subcore")
    k = pl.kernel(
        _sc_agg_body,
        out_type=jax.ShapeDtypeStruct((NC, NPAD, D), jnp.float32),
        mesh=mesh,
        scratch_types=(
            [pltpu.VMEM((CHUNK,), jnp.int32)] * 4
            + [pltpu.VMEM((CHUNK, D), jnp.float32)] * 2
            + [pltpu.VMEM_SHARED((NPAD, D), jnp.float32)]
            + [pltpu.SemaphoreType.DMA] * 6
        ),
    )
    return k(h, rowp, colp)


BLK = 1000
GRID = N // BLK


def _stage_a_body(x_ref, wt_ref, bt_ref, w1_ref, xt_ref, h1_ref):
    xt = jnp.dot(x_ref[...], wt_ref[...], precision=_HIGH,
                 preferred_element_type=jnp.float32) + bt_ref[...]
    xt_ref[...] = xt
    h1_ref[...] = jnp.dot(xt, w1_ref[...], precision=_HIGH,
                          preferred_element_type=jnp.float32)


def _stage_mid_body(xp_ref, p_ref, b_ref, w_ref, xn_ref, hn_ref):
    agg = p_ref[0] + p_ref[1] + b_ref[...]
    xn = xp_ref[...] + jnp.maximum(agg, 0.0)
    xn_ref[...] = xn
    hn_ref[...] = jnp.dot(xn, w_ref[...], precision=_HIGH,
                          preferred_element_type=jnp.float32)


def _stage_out_body(xp_ref, p_ref, b_ref, o_ref):
    agg = p_ref[0] + p_ref[1] + b_ref[...]
    o_ref[...] = xp_ref[...] + jnp.maximum(agg, 0.0)


_row_spec = pl.BlockSpec((BLK, D), lambda i: (i, 0))
_mat_spec = pl.BlockSpec((D, D), lambda i: (0, 0))
_vec_spec = pl.BlockSpec((1, D), lambda i: (0, 0))
_par_spec = pl.BlockSpec((NC, BLK, D), lambda i: (0, i, 0))
_rowD = jax.ShapeDtypeStruct((N, D), jnp.float32)


def _stage_a(x, wt, bt, w1):
    return pl.pallas_call(
        _stage_a_body,
        grid=(GRID,),
        in_specs=[_row_spec, _mat_spec, _vec_spec, _mat_spec],
        out_specs=[_row_spec, _row_spec],
        out_shape=[_rowD, _rowD],
    )(x, wt, bt, w1)


def _stage_mid(xp, p, b, w):
    return pl.pallas_call(
        _stage_mid_body,
        grid=(GRID,),
        in_specs=[_row_spec, _par_spec, _vec_spec, _mat_spec],
        out_specs=[_row_spec, _row_spec],
        out_shape=[_rowD, _rowD],
    )(xp, p, b, w)


def _stage_out(xp, p, b):
    return pl.pallas_call(
        _stage_out_body,
        grid=(GRID,),
        in_specs=[_row_spec, _par_spec, _vec_spec],
        out_specs=_row_spec,
        out_shape=_rowD,
    )(xp, p, b)


@jax.jit
def kernel(x, edge_index, Wt, bt, W1, b1, W2, b2):
    row = edge_index[0]
    col = edge_index[1]
    npad = EPAD - E
    rowp = jnp.concatenate([row, jnp.full((npad,), N, jnp.int32)])
    colp = jnp.concatenate([col, jnp.zeros((npad,), jnp.int32)])
    bt2 = bt.reshape(1, D)
    b12 = b1.reshape(1, D)
    b22 = b2.reshape(1, D)

    xt, h1 = _stage_a(x, Wt, bt2, W1)
    p1 = _sc_aggregate(h1, rowp, colp)
    x1, h2 = _stage_mid(xt, p1, b12, W2)
    p2 = _sc_aggregate(h2, rowp, colp)
    out = _stage_out(x1, p2, b22)
    return (out, jnp.float32(0.0))
